# Initial kernel scaffold; baseline (speedup 1.0000x reference)
import jax
import jax.numpy as jnp
from jax.experimental import pallas as pl


def kernel(x_seq, h0, w1t, b1, w_ih_t, b_ih, w_hh_t, b_hh, w2t, b2):
    raise NotImplementedError("write your pallas kernel here")


# speedup vs baseline: 13.9695x; 13.9695x over previous
import jax
import jax.numpy as jnp
from jax.experimental import pallas as pl
from jax.experimental.pallas import tpu as pltpu


def _rollout_kernel(x_ref, h0_ref, w1_ref, b1_ref, wih_ref, bih_ref,
                    whh_ref, bhh_ref, w2_ref, b2_ref,
                    q_ref, hout_ref, h_scr, *, chunk):
    t = pl.program_id(1)
    H = h_scr.shape[-1]

    @pl.when(t == 0)
    def _():
        h_scr[...] = h0_ref[...]

    h = h_scr[...]
    w1 = w1_ref[...]
    wih = wih_ref[...]
    whh = whh_ref[...]
    w2 = w2_ref[...]
    b1 = b1_ref[...]
    bih = bih_ref[...]
    bhh = bhh_ref[...]
    b2 = b2_ref[...]

    for i in range(chunk):
        x = x_ref[i].astype(jnp.bfloat16)
        a = jnp.dot(x, w1, preferred_element_type=jnp.float32) + b1
        a = jnp.maximum(a, 0.0).astype(jnp.bfloat16)
        gi = jnp.dot(a, wih, preferred_element_type=jnp.float32) + bih
        gh = jnp.dot(h.astype(jnp.bfloat16), whh,
                     preferred_element_type=jnp.float32) + bhh
        r = jax.nn.sigmoid(gi[:, 0:H] + gh[:, 0:H])
        z = jax.nn.sigmoid(gi[:, H:2 * H] + gh[:, H:2 * H])
        n = jnp.tanh(gi[:, 2 * H:3 * H] + r * gh[:, 2 * H:3 * H])
        h = (1.0 - z) * n + z * h
        q_ref[i] = jnp.dot(h.astype(jnp.bfloat16), w2,
                           preferred_element_type=jnp.float32) + b2

    h_scr[...] = h
    hout_ref[...] = h


def _rollout(x_seq, h0, w1t, b1, w_ih_t, b_ih, w_hh_t, b_hh, w2t, b2,
             *, block_b, chunk):
    T, B, in_dim = x_seq.shape
    H = h0.shape[1]
    A = w2t.shape[1]
    assert B % block_b == 0 and T % chunk == 0

    bf = jnp.bfloat16
    w1b, wihb, whhb, w2b = (w.astype(bf) for w in (w1t, w_ih_t, w_hh_t, w2t))

    def wspec(arr):
        return pl.BlockSpec(arr.shape, lambda b, t: (0,) * arr.ndim)

    grid = (B // block_b, T // chunk)
    import functools
    body = functools.partial(_rollout_kernel, chunk=chunk)

    q_seq, h_final = pl.pallas_call(
        body,
        out_shape=(
            jax.ShapeDtypeStruct((T, B, A), jnp.float32),
            jax.ShapeDtypeStruct((B, H), jnp.float32),
        ),
        grid_spec=pltpu.PrefetchScalarGridSpec(
            num_scalar_prefetch=0,
            grid=grid,
            in_specs=[
                pl.BlockSpec((chunk, block_b, in_dim), lambda b, t: (t, b, 0)),
                pl.BlockSpec((block_b, H), lambda b, t: (b, 0)),
                wspec(w1b), wspec(b1),
                wspec(wihb), wspec(b_ih),
                wspec(whhb), wspec(b_hh),
                wspec(w2b), wspec(b2),
            ],
            out_specs=(
                pl.BlockSpec((chunk, block_b, A), lambda b, t: (t, b, 0)),
                pl.BlockSpec((block_b, H), lambda b, t: (b, 0)),
            ),
            scratch_shapes=[pltpu.VMEM((block_b, H), jnp.float32)],
        ),
        compiler_params=pltpu.CompilerParams(
            dimension_semantics=("parallel", "arbitrary")),
    )(
        x_seq, h0,
        w1b, b1, wihb, b_ih, whhb, b_hh, w2b, b2,
    )
    return q_seq, h_final


def kernel(x_seq, h0, w1t, b1, w_ih_t, b_ih, w_hh_t, b_hh, w2t, b2):
    return _rollout(x_seq, h0, w1t, b1, w_ih_t, b_ih, w_hh_t, b_hh, w2t, b2,
                    block_b=128, chunk=8)


# speedup vs baseline: 16.7394x; 1.1983x over previous
import jax
import jax.numpy as jnp
from jax.experimental import pallas as pl
from jax.experimental.pallas import tpu as pltpu


def _rollout_kernel(x_ref, h0_ref, w1_ref, b1_ref, wih_ref, bih_ref,
                    whh_ref, bhh_ref, w2_ref, b2_ref,
                    q_ref, hout_ref, h_scr, *, chunk):
    t = pl.program_id(1)
    H = h_scr.shape[-1]

    @pl.when(t == 0)
    def _():
        h_scr[...] = h0_ref[...]

    h = h_scr[...]
    w1 = w1_ref[...]
    wih = wih_ref[...]
    whh = whh_ref[...]
    w2 = w2_ref[...]
    b1 = b1_ref[...]
    bih = bih_ref[...]
    bhh = bhh_ref[...]
    b2 = b2_ref[...]

    for i in range(chunk):
        x = x_ref[i].astype(jnp.bfloat16)
        a = jnp.dot(x, w1, preferred_element_type=jnp.float32) + b1
        a = jnp.maximum(a, 0.0).astype(jnp.bfloat16)
        gi = jnp.dot(a, wih, preferred_element_type=jnp.float32) + bih
        gh = jnp.dot(h.astype(jnp.bfloat16), whh,
                     preferred_element_type=jnp.float32) + bhh
        r = jax.nn.sigmoid(gi[:, 0:H] + gh[:, 0:H])
        z = jax.nn.sigmoid(gi[:, H:2 * H] + gh[:, H:2 * H])
        n = jnp.tanh(gi[:, 2 * H:3 * H] + r * gh[:, 2 * H:3 * H])
        h = (1.0 - z) * n + z * h
        q_ref[i] = jnp.dot(h.astype(jnp.bfloat16), w2,
                           preferred_element_type=jnp.float32) + b2

    h_scr[...] = h
    hout_ref[...] = h


def _rollout(x_seq, h0, w1t, b1, w_ih_t, b_ih, w_hh_t, b_hh, w2t, b2,
             *, block_b, chunk):
    T, B, in_dim = x_seq.shape
    H = h0.shape[1]
    A = w2t.shape[1]
    assert B % block_b == 0 and T % chunk == 0

    bf = jnp.bfloat16
    w1b, wihb, whhb, w2b = (w.astype(bf) for w in (w1t, w_ih_t, w_hh_t, w2t))

    def wspec(arr):
        return pl.BlockSpec(arr.shape, lambda b, t: (0,) * arr.ndim)

    grid = (B // block_b, T // chunk)
    import functools
    body = functools.partial(_rollout_kernel, chunk=chunk)

    q_seq, h_final = pl.pallas_call(
        body,
        out_shape=(
            jax.ShapeDtypeStruct((T, B, A), jnp.float32),
            jax.ShapeDtypeStruct((B, H), jnp.float32),
        ),
        grid_spec=pltpu.PrefetchScalarGridSpec(
            num_scalar_prefetch=0,
            grid=grid,
            in_specs=[
                pl.BlockSpec((chunk, block_b, in_dim), lambda b, t: (t, b, 0)),
                pl.BlockSpec((block_b, H), lambda b, t: (b, 0)),
                wspec(w1b), wspec(b1),
                wspec(wihb), wspec(b_ih),
                wspec(whhb), wspec(b_hh),
                wspec(w2b), wspec(b2),
            ],
            out_specs=(
                pl.BlockSpec((chunk, block_b, A), lambda b, t: (t, b, 0)),
                pl.BlockSpec((block_b, H), lambda b, t: (b, 0)),
            ),
            scratch_shapes=[pltpu.VMEM((block_b, H), jnp.float32)],
        ),
        compiler_params=pltpu.CompilerParams(
            dimension_semantics=("parallel", "arbitrary")),
    )(
        x_seq, h0,
        w1b, b1, wihb, b_ih, whhb, b_hh, w2b, b2,
    )
    return q_seq, h_final


def kernel(x_seq, h0, w1t, b1, w_ih_t, b_ih, w_hh_t, b_hh, w2t, b2):
    return _rollout(x_seq, h0, w1t, b1, w_ih_t, b_ih, w_hh_t, b_hh, w2t, b2,
                    block_b=256, chunk=8)
